# baseline (speedup 1.0000x reference)
import jax
import jax.numpy as jnp
from jax import lax
from jax.experimental import pallas as pl
from jax.experimental.pallas import tpu as pltpu
from jax.experimental.pallas import tpu_sc as plsc

BATCH = 16384
NCARDS = 7
DIM = 128
NCHUNK = DIM // 16

NC = 2
NS = 16
NW = NC * NS
BPW = BATCH // NW
XPW = BPW * NCARDS


def _sc_body(x_hbm, card_hbm, rank_hbm, suit_hbm, out_hbm,
             card_v, rank_v, suit_v, t_v, x_v, out_v):
    wid = lax.axis_index("s") * NC + lax.axis_index("c")
    pltpu.sync_copy(card_hbm, card_v)
    pltpu.sync_copy(rank_hbm, rank_v)
    pltpu.sync_copy(suit_hbm, suit_v)
    pltpu.sync_copy(x_hbm.at[pl.ds(wid * XPW, XPW)], x_v.at[pl.ds(0, XPW)])

    @plsc.parallel_loop(1, 52)
    def _build(i):
        q = i // 4
        m = i - q * 4
        for j in range(NCHUNK):
            js = pl.ds(j * 16, 16)
            t_v[pl.ds(i * DIM + j * 16, 16)] = (
                card_v[i, js] + rank_v[q, js] + suit_v[m, js])

    zero = jnp.zeros((16,), jnp.float32)
    for j in range(NCHUNK):
        t_v[pl.ds(j * 16, 16)] = zero

    @plsc.parallel_loop(0, BPW, unroll=2)
    def _row(b):
        xrow = x_v[pl.ds(b * NCARDS, 16)]
        base = [xrow[c] * DIM for c in range(NCARDS)]
        for j in range(NCHUNK):
            js = j * 16
            t0 = t_v[pl.ds(base[0] + js, 16)]
            t1 = t_v[pl.ds(base[1] + js, 16)]
            t2 = t_v[pl.ds(base[2] + js, 16)]
            t3 = t_v[pl.ds(base[3] + js, 16)]
            t4 = t_v[pl.ds(base[4] + js, 16)]
            t5 = t_v[pl.ds(base[5] + js, 16)]
            t6 = t_v[pl.ds(base[6] + js, 16)]
            out_v[b, pl.ds(js, 16)] = ((t0 + t1) + (t2 + t3)) + ((t4 + t5) + t6)

    pltpu.sync_copy(out_v, out_hbm.at[pl.ds(wid * BPW, BPW)])


@jax.jit
def kernel(X, card, rank, suit):
    f = pl.kernel(
        _sc_body,
        out_type=jax.ShapeDtypeStruct((BATCH, DIM), jnp.float32),
        mesh=plsc.VectorSubcoreMesh(core_axis_name="c", subcore_axis_name="s"),
        compiler_params=pltpu.CompilerParams(needs_layout_passes=False),
        scratch_types=[
            pltpu.VMEM((52, DIM), jnp.float32),
            pltpu.VMEM((13, DIM), jnp.float32),
            pltpu.VMEM((4, DIM), jnp.float32),
            pltpu.VMEM((52 * DIM,), jnp.float32),
            pltpu.VMEM((XPW + 16,), jnp.int32),
            pltpu.VMEM((BPW, DIM), jnp.float32),
        ],
    )
    return f(X.reshape(-1).astype(jnp.int32), card.astype(jnp.float32),
             rank.astype(jnp.float32), suit.astype(jnp.float32))


# speedup vs baseline: 1.0367x; 1.0367x over previous
import jax
import jax.numpy as jnp
from jax import lax
from jax.experimental import pallas as pl
from jax.experimental.pallas import tpu as pltpu
from jax.experimental.pallas import tpu_sc as plsc

BATCH = 16384
NCARDS = 7
DIM = 128
NCHUNK = DIM // 16

NC = 2
NS = 16
NW = NC * NS
BPW = BATCH // NW
CROWS = 128
NCHUNKS = BPW // CROWS


def _sc_body(x_hbm, card_hbm, rank_hbm, suit_hbm, out_hbm,
             card_v, rank_v, suit_v, t_v,
             x0_v, x1_v, o0_v, o1_v,
             sx0, sx1, so0, so1):
    wid = lax.axis_index("s") * NC + lax.axis_index("c")
    row0 = wid * BPW
    xbufs, obufs = (x0_v, x1_v), (o0_v, o1_v)
    xsems, osems = (sx0, sx1), (so0, so1)

    xd = [None] * NCHUNKS
    xd[0] = pltpu.async_copy(x_hbm.at[pl.ds(row0, CROWS)], xbufs[0], xsems[0])

    pltpu.sync_copy(card_hbm, card_v)
    pltpu.sync_copy(rank_hbm, rank_v)
    pltpu.sync_copy(suit_hbm, suit_v)

    @plsc.parallel_loop(1, 52)
    def _build(i):
        q = i // 4
        m = i - q * 4
        for j in range(NCHUNK):
            js = pl.ds(j * 16, 16)
            t_v[pl.ds(i * DIM + j * 16, 16)] = (
                card_v[i, js] + rank_v[q, js] + suit_v[m, js])

    zero = jnp.zeros((16,), jnp.float32)
    for j in range(NCHUNK):
        t_v[pl.ds(j * 16, 16)] = zero

    od = [None] * NCHUNKS
    for k in range(NCHUNKS):
        xb, ob = xbufs[k % 2], obufs[k % 2]
        if k + 1 < NCHUNKS:
            xd[k + 1] = pltpu.async_copy(
                x_hbm.at[pl.ds(row0 + (k + 1) * CROWS, CROWS)],
                xbufs[(k + 1) % 2], xsems[(k + 1) % 2])
        xd[k].wait()
        if k >= 2:
            od[k - 2].wait()

        @plsc.parallel_loop(0, CROWS, unroll=2)
        def _row(b):
            xrow = xb[b, pl.ds(0, 16)]
            base = [xrow[c] * DIM for c in range(NCARDS)]
            for j in range(NCHUNK):
                js = j * 16
                t0 = t_v[pl.ds(base[0] + js, 16)]
                t1 = t_v[pl.ds(base[1] + js, 16)]
                t2 = t_v[pl.ds(base[2] + js, 16)]
                t3 = t_v[pl.ds(base[3] + js, 16)]
                t4 = t_v[pl.ds(base[4] + js, 16)]
                t5 = t_v[pl.ds(base[5] + js, 16)]
                t6 = t_v[pl.ds(base[6] + js, 16)]
                ob[b, pl.ds(js, 16)] = ((t0 + t1) + (t2 + t3)) + ((t4 + t5) + t6)

        od[k] = pltpu.async_copy(
            ob, out_hbm.at[pl.ds(row0 + k * CROWS, CROWS)], osems[k % 2])
    od[NCHUNKS - 2].wait()
    od[NCHUNKS - 1].wait()


@jax.jit
def kernel(X, card, rank, suit):
    xp = jnp.pad(X.astype(jnp.int32), ((0, 0), (0, DIM - NCARDS)))
    f = pl.kernel(
        _sc_body,
        out_type=jax.ShapeDtypeStruct((BATCH, DIM), jnp.float32),
        mesh=plsc.VectorSubcoreMesh(core_axis_name="c", subcore_axis_name="s"),
        compiler_params=pltpu.CompilerParams(needs_layout_passes=False),
        scratch_types=[
            pltpu.VMEM((52, DIM), jnp.float32),
            pltpu.VMEM((13, DIM), jnp.float32),
            pltpu.VMEM((4, DIM), jnp.float32),
            pltpu.VMEM((52 * DIM,), jnp.float32),
            pltpu.VMEM((CROWS, DIM), jnp.int32),
            pltpu.VMEM((CROWS, DIM), jnp.int32),
            pltpu.VMEM((CROWS, DIM), jnp.float32),
            pltpu.VMEM((CROWS, DIM), jnp.float32),
            pltpu.SemaphoreType.DMA,
            pltpu.SemaphoreType.DMA,
            pltpu.SemaphoreType.DMA,
            pltpu.SemaphoreType.DMA,
        ],
    )
    return f(xp, card.astype(jnp.float32),
             rank.astype(jnp.float32), suit.astype(jnp.float32))
